# Initial kernel scaffold; baseline (speedup 1.0000x reference)
import jax
import jax.numpy as jnp
from jax.experimental import pallas as pl


def kernel(pred, target, allowed_classes):
    raise NotImplementedError("write your pallas kernel here")


# speedup vs baseline: 6.3475x; 6.3475x over previous
import functools

import jax
import jax.numpy as jnp
from jax import lax
from jax.experimental import pallas as pl
from jax.experimental.pallas import tpu as pltpu
from jax.experimental.pallas import tpu_sc as plsc

_EPS = 0.1
_PAD = -1


def _sc_gather(pred_flat, idx3):
    NW, NCH, L = idx3.shape
    info = plsc.get_sparse_core_info()
    nc = info.num_cores
    mesh = plsc.VectorSubcoreMesh(core_axis_name="c", subcore_axis_name="s")

    @functools.partial(
        pl.kernel,
        out_type=jax.ShapeDtypeStruct((NW, NCH, L), jnp.float32),
        mesh=mesh,
        scratch_types=[
            pltpu.VMEM((NCH, L), jnp.int32),
            pltpu.VMEM((NCH, L), jnp.float32),
            pltpu.SemaphoreType.DMA,
        ],
    )
    def gk(pred_hbm, idx_hbm, out_hbm, idx_v, rows_v, sem):
        wid = lax.axis_index("s") * nc + lax.axis_index("c")
        pltpu.sync_copy(idx_hbm.at[wid], idx_v)
        cps = [
            pltpu.async_copy(pred_hbm.at[idx_v.at[j]], rows_v.at[j], sem)
            for j in range(NCH)
        ]
        for cp in cps:
            cp.wait()
        pltpu.sync_copy(rows_v, out_hbm.at[wid])

    return gk(pred_flat, idx3)


def _lse(pred):
    B, C = pred.shape
    RB, CB = 256, 8192
    nI = B // RB
    nJ = pl.cdiv(C, CB)

    def body(x_ref, o_ref, m_s, s_s):
        j = pl.program_id(1)

        @pl.when(j == 0)
        def _():
            m_s[...] = jnp.full_like(m_s, -jnp.inf)
            s_s[...] = jnp.zeros_like(s_s)

        x = x_ref[...]
        cols = j * CB + lax.broadcasted_iota(jnp.int32, x.shape, 1)
        x = jnp.where(cols < C, x, -jnp.inf)
        bm = jnp.max(x, axis=1, keepdims=True)
        m_old = jnp.max(m_s[...], axis=1, keepdims=True)
        s_old = jnp.max(s_s[...], axis=1, keepdims=True)
        m_new = jnp.maximum(m_old, bm)
        s_new = s_old * jnp.exp(m_old - m_new) + jnp.sum(
            jnp.exp(x - m_new), axis=1, keepdims=True
        )
        m_s[...] = jnp.broadcast_to(m_new, m_s.shape)
        s_s[...] = jnp.broadcast_to(s_new, s_s.shape)

        @pl.when(j == nJ - 1)
        def _():
            o_ref[...] = jnp.broadcast_to(m_new + jnp.log(s_new), o_ref.shape)

    return pl.pallas_call(
        body,
        grid=(nI, nJ),
        in_specs=[pl.BlockSpec((RB, CB), lambda i, j: (i, j))],
        out_specs=pl.BlockSpec((RB, 128), lambda i, j: (i, 0)),
        out_shape=jax.ShapeDtypeStruct((B, 128), jnp.float32),
        scratch_shapes=[
            pltpu.VMEM((RB, 128), jnp.float32),
            pltpu.VMEM((RB, 128), jnp.float32),
        ],
    )(pred)


def _combine(cls_t, g_t, lse3, B, K):
    KP = cls_t.shape[0]
    nblk = B // 128

    def body(c_ref, gref, l_ref, o_ref):
        i = pl.program_id(0)
        a = c_ref[...]
        g = gref[...]
        lse = l_ref[0]
        a50 = a[:K]
        t = a[K : K + 1]
        ga = g[:K]
        gt = g[K : K + 1]
        valid = a50 != _PAD
        eq = a50[:, None, :] == a50[None, :, :]
        ji = lax.broadcasted_iota(jnp.int32, (K, K, 128), 0)
        ki = lax.broadcasted_iota(jnp.int32, (K, K, 128), 1)
        dup = jnp.any(eq & (ji < ki), axis=0)
        contrib = valid & (~dup) & (a50 != t)
        cf = contrib.astype(jnp.float32)
        cnt = jnp.sum(cf, axis=0, keepdims=True)
        nv = jnp.sum(valid.astype(jnp.float32), axis=0, keepdims=True)
        e = _EPS / nv
        tmass = (1.0 - _EPS) + cnt * e
        const = cnt * e * jnp.log(e) + (1.0 - _EPS) * jnp.log(1.0 - _EPS)
        sum_wg = jnp.sum(cf * ga, axis=0, keepdims=True)
        row = tmass * lse + const - e * sum_wg - (1.0 - _EPS) * gt
        partial = jnp.sum(row) / B

        @pl.when(i == 0)
        def _():
            o_ref[...] = jnp.zeros_like(o_ref)

        o_ref[...] += partial.reshape(1, 1)

    out = pl.pallas_call(
        body,
        grid=(nblk,),
        in_specs=[
            pl.BlockSpec((KP, 128), lambda i: (0, i)),
            pl.BlockSpec((KP, 128), lambda i: (0, i)),
            pl.BlockSpec((1, 1, 128), lambda i: (i, 0, 0)),
        ],
        out_specs=pl.BlockSpec((1, 1), lambda i: (0, 0)),
        out_shape=jax.ShapeDtypeStruct((1, 1), jnp.float32),
    )(cls_t, g_t, lse3)
    return out[0, 0]


def kernel(pred, target, allowed_classes):
    B, C = pred.shape
    K = allowed_classes.shape[1]
    target = target.astype(jnp.int32)
    allowed = allowed_classes.astype(jnp.int32)
    KP = -(-(K + 1) // 8) * 8
    pad = jnp.zeros((B, KP - K - 1), jnp.int32)
    cls = jnp.concatenate([allowed, target[:, None], pad], axis=1)
    safe = jnp.where(cls == _PAD, 0, cls)
    flat_idx = jnp.arange(B, dtype=jnp.int32)[:, None] * C + safe
    NW = 32
    idx3 = flat_idx.reshape(NW, (B * KP) // (NW * 128), 128)
    g = _sc_gather(pred.reshape(-1), idx3)
    gathered = g.reshape(B, KP)
    lse2d = _lse(pred)
    lse3 = lse2d[:, 0].reshape(B // 128, 1, 128)
    return _combine(cls.T, gathered.T, lse3, B, K)
